# Initial kernel scaffold; baseline (speedup 1.0000x reference)
import jax
import jax.numpy as jnp
from jax.experimental import pallas as pl


def kernel(input, edge, symm_update, W1, b1, W2, b2):
    raise NotImplementedError("write your pallas kernel here")


# speedup vs baseline: 2.3651x; 2.3651x over previous
import dataclasses
import functools

import jax
import jax.numpy as jnp
from jax import lax
from jax.experimental import pallas as pl
from jax.experimental.pallas import tpu as pltpu
from jax.experimental.pallas import tpu_sc as plsc

N = 10000
E = 160000
C = 256
H = 128
NT = 16
EPT = E // NT
CHUNK = 80
NCH = EPT // CHUNK
RPT = 624
RTAIL = N - NT * RPT
GROUPS = H // 16

BN = 1000
NB = N // BN


def _mm1_body(x_ref, w_ref, b_ref, o_ref):
    o_ref[...] = (
        jnp.dot(x_ref[...], w_ref[...], preferred_element_type=jnp.float32)
        + b_ref[0]
    )


def _mm1(x, w, b):
    b2 = b.reshape(2, 1, H)
    return pl.pallas_call(
        _mm1_body,
        grid=(NB, 2),
        in_specs=[
            pl.BlockSpec((BN, C), lambda i, c: (i, 0)),
            pl.BlockSpec((C, H), lambda i, c: (0, c)),
            pl.BlockSpec((1, 1, H), lambda i, c: (c, 0, 0)),
        ],
        out_specs=pl.BlockSpec((BN, H), lambda i, c: (c * NB + i, 0)),
        out_shape=jax.ShapeDtypeStruct((2 * N, H), jnp.float32),
    )(x, w, b2)


def _mm2_body(a0_ref, a1_ref, w_ref, b_ref, o_ref):
    x0 = jnp.maximum(a0_ref[...], 0.0)
    x1 = jnp.maximum(a1_ref[...], 0.0)
    o_ref[...] = (
        jnp.dot(x0, w_ref[:H, :], preferred_element_type=jnp.float32)
        + jnp.dot(x1, w_ref[H:, :], preferred_element_type=jnp.float32)
        + b_ref[0]
    )


def _mm2(agg, w, b):
    b2 = b.reshape(2, 1, H)
    return pl.pallas_call(
        _mm2_body,
        grid=(NB, 2),
        in_specs=[
            pl.BlockSpec((BN, H), lambda i, c: (i, 0)),
            pl.BlockSpec((BN, H), lambda i, c: (NB + i, 0)),
            pl.BlockSpec((C, H), lambda i, c: (0, c)),
            pl.BlockSpec((1, 1, H), lambda i, c: (c, 0, 0)),
        ],
        out_specs=pl.BlockSpec((BN, H), lambda i, c: (c * NB + i, 0)),
        out_shape=jax.ShapeDtypeStruct((2 * N, H), jnp.float32),
    )(agg, agg, w, b2)


def _epilogue_body(in_ref, a0_ref, a1_ref, o_ref):
    o_ref[:, :H] = (in_ref[:, :H] + jnp.maximum(a0_ref[...], 0.0)) * 0.5
    o_ref[:, H:] = (in_ref[:, H:] + jnp.maximum(a1_ref[...], 0.0)) * 0.5


def _epilogue(inp, agg):
    return pl.pallas_call(
        _epilogue_body,
        grid=(NB,),
        in_specs=[
            pl.BlockSpec((BN, C), lambda i: (i, 0)),
            pl.BlockSpec((BN, H), lambda i: (i, 0)),
            pl.BlockSpec((BN, H), lambda i: (NB + i, 0)),
        ],
        out_specs=pl.BlockSpec((BN, C), lambda i: (i, 0)),
        out_shape=jax.ShapeDtypeStruct((N, C), jnp.float32),
    )(inp, agg, agg)


def _sc_agg_kernel(src_hbm, dst_hbm, w_hbm, h_hbm, out_hbm,
                   se_v, de_v, we_v, buf, acc):
    c = lax.axis_index("c")
    s = lax.axis_index("s")

    zero = jnp.zeros((16,), jnp.float32)

    @pl.loop(0, CHUNK)
    def _zero_buf(r):
        for g in range(GROUPS):
            buf[r, pl.ds(g * 16, 16)] = zero

    base = s * RPT
    for k in range(RPT // CHUNK):
        pltpu.sync_copy(buf, acc.at[pl.ds(base + k * CHUNK, CHUNK)])
    rem = RPT % CHUNK
    if rem:
        pltpu.sync_copy(buf.at[pl.ds(0, rem)],
                        acc.at[pl.ds(base + (RPT // CHUNK) * CHUNK, rem)])

    @pl.when(s == NT - 1)
    def _zero_tail():
        pltpu.sync_copy(buf.at[pl.ds(0, RTAIL)],
                        acc.at[pl.ds(NT * RPT, RTAIL)])

    plsc.subcore_barrier()

    @pl.loop(0, NCH)
    def _window(j):
        pltpu.sync_copy(src_hbm.at[c, s, j], se_v.at[0])
        pltpu.sync_copy(dst_hbm.at[s, j], de_v.at[0])
        pltpu.sync_copy(w_hbm.at[s, j], we_v.at[0])
        pltpu.sync_copy(h_hbm.at[se_v.at[0]], buf)

        @pl.loop(0, CHUNK)
        def _edge(r):
            wspl = plsc.load_gather(
                we_v, [jnp.zeros((16,), jnp.int32),
                       jnp.full((16,), r, jnp.int32)])
            for g in range(GROUPS):
                sl = pl.ds(g * 16, 16)
                buf[r, sl] = buf[r, sl] * wspl

        pltpu.sync_copy(buf, acc.at[de_v.at[0]], add=True)

    plsc.subcore_barrier()

    pltpu.sync_copy(acc.at[pl.ds(base, RPT)],
                    out_hbm.at[pl.ds(c * N + base, RPT)])

    @pl.when(s == NT - 1)
    def _write_tail():
        pltpu.sync_copy(acc.at[pl.ds(NT * RPT, RTAIL)],
                        out_hbm.at[pl.ds(c * N + NT * RPT, RTAIL)])


def _sc_compiler_params():
    cp = pltpu.CompilerParams()
    if "needs_layout_passes" in pltpu.CompilerParams.__dataclass_fields__:
        cp = dataclasses.replace(cp, needs_layout_passes=False)
    return cp


def _sc_agg(src_c, dst_t, w_t, h):
    mesh = plsc.VectorSubcoreMesh(core_axis_name="c", subcore_axis_name="s")
    kern = functools.partial(
        pl.kernel,
        mesh=mesh,
        compiler_params=_sc_compiler_params(),
        out_type=jax.ShapeDtypeStruct((2 * N, H), jnp.float32),
        scratch_types=[
            pltpu.VMEM((1, CHUNK), jnp.int32),
            pltpu.VMEM((1, CHUNK), jnp.int32),
            pltpu.VMEM((1, CHUNK), jnp.float32),
            pltpu.VMEM((CHUNK, H), jnp.float32),
            pltpu.VMEM_SHARED((N, H), jnp.float32),
        ],
    )(_sc_agg_kernel)
    return kern(src_c, dst_t, w_t, h)


@jax.jit
def _run(input, edge, symm_update, W1, b1, W2, b2):
    src = edge[0]
    dst = edge[1]
    src_c = jnp.stack([src, src + N]).reshape(2, NT, NCH, CHUNK)
    dst_t = dst.reshape(NT, NCH, CHUNK)
    w_t = symm_update.reshape(NT, NCH, CHUNK)

    h1 = _mm1(input, W1, b1)
    agg1 = _sc_agg(src_c, dst_t, w_t, h1)
    h2 = _mm2(agg1, W2, b2)
    agg2 = _sc_agg(src_c, dst_t, w_t, h2)
    return _epilogue(input, agg2)


def kernel(input, edge, symm_update, W1, b1, W2, b2):
    return _run(input, edge, symm_update, W1, b1, W2, b2)


# speedup vs baseline: 2.5501x; 1.0782x over previous
import dataclasses
import functools

import jax
import jax.numpy as jnp
from jax import lax
from jax.experimental import pallas as pl
from jax.experimental.pallas import tpu as pltpu
from jax.experimental.pallas import tpu_sc as plsc

N = 10000
E = 160000
C = 256
H = 128
NT = 16
EPT = E // NT
CHUNK = 80
NCH = EPT // CHUNK
RPT = 624
RTAIL = N - NT * RPT
GROUPS = H // 16

BN = 1000
NB = N // BN


def _mm1_body(x_ref, w_ref, b_ref, o_ref):
    o_ref[...] = (
        jnp.dot(x_ref[...], w_ref[...], preferred_element_type=jnp.float32)
        + b_ref[0]
    )


def _mm1(x, w, b):
    b2 = b.reshape(2, 1, H)
    return pl.pallas_call(
        _mm1_body,
        grid=(NB, 2),
        in_specs=[
            pl.BlockSpec((BN, C), lambda i, c: (i, 0)),
            pl.BlockSpec((C, H), lambda i, c: (0, c)),
            pl.BlockSpec((1, 1, H), lambda i, c: (c, 0, 0)),
        ],
        out_specs=pl.BlockSpec((BN, H), lambda i, c: (c * NB + i, 0)),
        out_shape=jax.ShapeDtypeStruct((2 * N, H), jnp.float32),
    )(x, w, b2)


def _mm2_body(a0_ref, a1_ref, w_ref, b_ref, o_ref):
    x0 = jnp.maximum(a0_ref[...], 0.0)
    x1 = jnp.maximum(a1_ref[...], 0.0)
    o_ref[...] = (
        jnp.dot(x0, w_ref[:H, :], preferred_element_type=jnp.float32)
        + jnp.dot(x1, w_ref[H:, :], preferred_element_type=jnp.float32)
        + b_ref[0]
    )


def _mm2(agg, w, b):
    b2 = b.reshape(2, 1, H)
    return pl.pallas_call(
        _mm2_body,
        grid=(NB, 2),
        in_specs=[
            pl.BlockSpec((BN, H), lambda i, c: (i, 0)),
            pl.BlockSpec((BN, H), lambda i, c: (NB + i, 0)),
            pl.BlockSpec((C, H), lambda i, c: (0, c)),
            pl.BlockSpec((1, 1, H), lambda i, c: (c, 0, 0)),
        ],
        out_specs=pl.BlockSpec((BN, H), lambda i, c: (c * NB + i, 0)),
        out_shape=jax.ShapeDtypeStruct((2 * N, H), jnp.float32),
    )(agg, agg, w, b2)


def _epilogue_body(in_ref, a0_ref, a1_ref, o_ref):
    o_ref[:, :H] = (in_ref[:, :H] + jnp.maximum(a0_ref[...], 0.0)) * 0.5
    o_ref[:, H:] = (in_ref[:, H:] + jnp.maximum(a1_ref[...], 0.0)) * 0.5


def _epilogue(inp, agg):
    return pl.pallas_call(
        _epilogue_body,
        grid=(NB,),
        in_specs=[
            pl.BlockSpec((BN, C), lambda i: (i, 0)),
            pl.BlockSpec((BN, H), lambda i: (i, 0)),
            pl.BlockSpec((BN, H), lambda i: (NB + i, 0)),
        ],
        out_specs=pl.BlockSpec((BN, C), lambda i: (i, 0)),
        out_shape=jax.ShapeDtypeStruct((N, C), jnp.float32),
    )(inp, agg, agg)


NBUF = 3
NE = 2 * NBUF


def _sc_agg_kernel(src_hbm, dst_hbm, w_hbm, h_hbm, out_hbm,
                   se, de, we, rbuf, acc, esem, gsem, asem):
    c = lax.axis_index("c")
    s = lax.axis_index("s")

    def e_copies(j):
        q = lax.rem(j, NE)
        return (
            pltpu.make_async_copy(src_hbm.at[c, s, j], se.at[q], esem.at[q]),
            pltpu.make_async_copy(dst_hbm.at[s, j], de.at[q], esem.at[q]),
            pltpu.make_async_copy(w_hbm.at[s, j], we.at[q], esem.at[q]),
        )

    def e_start(j):
        for cp in e_copies(j):
            cp.start()

    def e_wait(j):
        for cp in e_copies(j):
            cp.wait()

    def g_desc(j):
        p = lax.rem(j, NBUF)
        q = lax.rem(j, NE)
        return pltpu.make_async_copy(h_hbm.at[se.at[q]], rbuf.at[p],
                                     gsem.at[p])

    def a_desc(j):
        p = lax.rem(j, NBUF)
        q = lax.rem(j, NE)
        return pltpu.make_async_copy(rbuf.at[p], acc.at[de.at[q]],
                                     asem.at[p])

    zero = jnp.zeros((16,), jnp.float32)

    @pl.loop(0, CHUNK)
    def _zero_buf(r):
        for g in range(GROUPS):
            rbuf[0, r, pl.ds(g * 16, 16)] = zero

    base = s * RPT
    for k in range(RPT // CHUNK):
        pltpu.sync_copy(rbuf.at[0], acc.at[pl.ds(base + k * CHUNK, CHUNK)])
    rem = RPT % CHUNK
    if rem:
        pltpu.sync_copy(rbuf.at[0].at[pl.ds(0, rem)],
                        acc.at[pl.ds(base + (RPT // CHUNK) * CHUNK, rem)])

    @pl.when(s == NT - 1)
    def _zero_tail():
        pltpu.sync_copy(rbuf.at[0].at[pl.ds(0, RTAIL)],
                        acc.at[pl.ds(NT * RPT, RTAIL)])

    plsc.subcore_barrier()

    for k in range(NBUF):
        e_start(jnp.int32(k))
    e_wait(jnp.int32(0))
    g_desc(jnp.int32(0)).start()

    @pl.loop(0, NCH)
    def _window(j):
        p = lax.rem(j, NBUF)
        q = lax.rem(j, NE)
        jn = j + 1

        @pl.when(jn < NCH)
        def _issue_next_gather():
            e_wait(jn)

            @pl.when(jn >= NBUF)
            def _():
                a_desc(jn - NBUF).wait()

            g_desc(jn).start()

        @pl.when(j + NBUF < NCH)
        def _prefetch_edges():
            e_start(j + NBUF)

        g_desc(j).wait()

        @pl.loop(0, CHUNK)
        def _edge(r):
            wspl = plsc.load_gather(
                we, [jnp.full((16,), q, jnp.int32),
                     jnp.full((16,), r, jnp.int32)])
            for g in range(GROUPS):
                sl = pl.ds(g * 16, 16)
                rbuf[p, r, sl] = rbuf[p, r, sl] * wspl

        a_desc(j).start(add=True)

    for k in range(NBUF):
        a_desc(jnp.int32(NCH - NBUF + k)).wait()

    plsc.subcore_barrier()

    pltpu.sync_copy(acc.at[pl.ds(base, RPT)],
                    out_hbm.at[pl.ds(c * N + base, RPT)])

    @pl.when(s == NT - 1)
    def _write_tail():
        pltpu.sync_copy(acc.at[pl.ds(NT * RPT, RTAIL)],
                        out_hbm.at[pl.ds(c * N + NT * RPT, RTAIL)])


def _sc_compiler_params():
    cp = pltpu.CompilerParams()
    if "needs_layout_passes" in pltpu.CompilerParams.__dataclass_fields__:
        cp = dataclasses.replace(cp, needs_layout_passes=False)
    return cp


def _sc_agg(src_c, dst_t, w_t, h):
    mesh = plsc.VectorSubcoreMesh(core_axis_name="c", subcore_axis_name="s")
    kern = functools.partial(
        pl.kernel,
        mesh=mesh,
        compiler_params=_sc_compiler_params(),
        out_type=jax.ShapeDtypeStruct((2 * N, H), jnp.float32),
        scratch_types=[
            pltpu.VMEM((NE, CHUNK), jnp.int32),
            pltpu.VMEM((NE, CHUNK), jnp.int32),
            pltpu.VMEM((NE, CHUNK), jnp.float32),
            pltpu.VMEM((NBUF, CHUNK, H), jnp.float32),
            pltpu.VMEM_SHARED((N, H), jnp.float32),
            pltpu.SemaphoreType.DMA((NE,)),
            pltpu.SemaphoreType.DMA((NBUF,)),
            pltpu.SemaphoreType.DMA((NBUF,)),
        ],
    )(_sc_agg_kernel)
    return kern(src_c, dst_t, w_t, h)


@jax.jit
def _run(input, edge, symm_update, W1, b1, W2, b2):
    src = edge[0]
    dst = edge[1]
    src_c = jnp.stack([src, src + N]).reshape(2, NT, NCH, CHUNK)
    dst_t = dst.reshape(NT, NCH, CHUNK)
    w_t = symm_update.reshape(NT, NCH, CHUNK)

    h1 = _mm1(input, W1, b1)
    agg1 = _sc_agg(src_c, dst_t, w_t, h1)
    h2 = _mm2(agg1, W2, b2)
    agg2 = _sc_agg(src_c, dst_t, w_t, h2)
    return _epilogue(input, agg2)


def kernel(input, edge, symm_update, W1, b1, W2, b2):
    return _run(input, edge, symm_update, W1, b1, W2, b2)


# speedup vs baseline: 7.1763x; 2.8141x over previous
import dataclasses
import functools

import jax
import jax.numpy as jnp
from jax import lax
from jax.experimental import pallas as pl
from jax.experimental.pallas import tpu as pltpu
from jax.experimental.pallas import tpu_sc as plsc

N = 10000
E = 160000
C = 256
H = 128
NT = 16
EPT = E // NT
CHUNK = 80
NCH = EPT // CHUNK
RPT = 624
RTAIL = N - NT * RPT
GROUPS = H // 16

BN = 1000
NB = N // BN


def _mm1_body(x_ref, w_ref, b_ref, o_ref):
    o_ref[...] = (
        jnp.dot(x_ref[...], w_ref[...], preferred_element_type=jnp.float32)
        + b_ref[0]
    )


def _mm1(x, w, b):
    b2 = b.reshape(2, 1, H)
    return pl.pallas_call(
        _mm1_body,
        grid=(NB, 2),
        in_specs=[
            pl.BlockSpec((BN, C), lambda i, c: (i, 0)),
            pl.BlockSpec((C, H), lambda i, c: (0, c)),
            pl.BlockSpec((1, 1, H), lambda i, c: (c, 0, 0)),
        ],
        out_specs=pl.BlockSpec((BN, H), lambda i, c: (c * NB + i, 0)),
        out_shape=jax.ShapeDtypeStruct((2 * N, H), jnp.float32),
    )(x, w, b2)


def _mm2_body(a0_ref, a1_ref, w_ref, b_ref, o_ref):
    x0 = jnp.maximum(a0_ref[...], 0.0)
    x1 = jnp.maximum(a1_ref[...], 0.0)
    o_ref[...] = (
        jnp.dot(x0, w_ref[:H, :], preferred_element_type=jnp.float32)
        + jnp.dot(x1, w_ref[H:, :], preferred_element_type=jnp.float32)
        + b_ref[0]
    )


def _mm2(agg, w, b):
    b2 = b.reshape(2, 1, H)
    return pl.pallas_call(
        _mm2_body,
        grid=(NB, 2),
        in_specs=[
            pl.BlockSpec((BN, H), lambda i, c: (i, 0)),
            pl.BlockSpec((BN, H), lambda i, c: (NB + i, 0)),
            pl.BlockSpec((C, H), lambda i, c: (0, c)),
            pl.BlockSpec((1, 1, H), lambda i, c: (c, 0, 0)),
        ],
        out_specs=pl.BlockSpec((BN, H), lambda i, c: (c * NB + i, 0)),
        out_shape=jax.ShapeDtypeStruct((2 * N, H), jnp.float32),
    )(agg, agg, w, b2)


def _epilogue_body(in_ref, a0_ref, a1_ref, o_ref):
    o_ref[:, :H] = (in_ref[:, :H] + jnp.maximum(a0_ref[...], 0.0)) * 0.5
    o_ref[:, H:] = (in_ref[:, H:] + jnp.maximum(a1_ref[...], 0.0)) * 0.5


def _epilogue(inp, agg):
    return pl.pallas_call(
        _epilogue_body,
        grid=(NB,),
        in_specs=[
            pl.BlockSpec((BN, C), lambda i: (i, 0)),
            pl.BlockSpec((BN, H), lambda i: (i, 0)),
            pl.BlockSpec((BN, H), lambda i: (NB + i, 0)),
        ],
        out_specs=pl.BlockSpec((BN, C), lambda i: (i, 0)),
        out_shape=jax.ShapeDtypeStruct((N, C), jnp.float32),
    )(inp, agg, agg)


NBUF = 3
NE = 2 * NBUF


def _sc_agg_kernel(src_hbm, dst_hbm, w_hbm, h_hbm, out_hbm,
                   se, de, we, rbuf, acc, esem, gsem, asem):
    c = lax.axis_index("c")
    s = lax.axis_index("s")

    def e_copies(j):
        q = lax.rem(j, NE)
        return (
            pltpu.make_async_copy(src_hbm.at[c, s, j], se.at[q], esem.at[q]),
            pltpu.make_async_copy(dst_hbm.at[s, j], de.at[q], esem.at[q]),
            pltpu.make_async_copy(w_hbm.at[s, j], we.at[q], esem.at[q]),
        )

    def e_start(j):
        for cp in e_copies(j):
            cp.start()

    def e_wait(j):
        for cp in e_copies(j):
            cp.wait()

    def g_desc(j):
        p = lax.rem(j, NBUF)
        q = lax.rem(j, NE)
        return pltpu.make_async_copy(h_hbm.at[se.at[q]], rbuf.at[p],
                                     gsem.at[p])

    def a_desc(j):
        p = lax.rem(j, NBUF)
        q = lax.rem(j, NE)
        return pltpu.make_async_copy(rbuf.at[p], acc.at[de.at[q]],
                                     asem.at[p])

    zero = jnp.zeros((16,), jnp.float32)

    @pl.loop(0, CHUNK)
    def _zero_buf(r):
        for g in range(GROUPS):
            rbuf[0, r, pl.ds(g * 16, 16)] = zero

    base = s * RPT
    for k in range(RPT // CHUNK):
        pltpu.sync_copy(rbuf.at[0], acc.at[pl.ds(base + k * CHUNK, CHUNK)])
    rem = RPT % CHUNK
    if rem:
        pltpu.sync_copy(rbuf.at[0].at[pl.ds(0, rem)],
                        acc.at[pl.ds(base + (RPT // CHUNK) * CHUNK, rem)])

    @pl.when(s == NT - 1)
    def _zero_tail():
        pltpu.sync_copy(rbuf.at[0].at[pl.ds(0, RTAIL)],
                        acc.at[pl.ds(NT * RPT, RTAIL)])

    plsc.subcore_barrier()

    for k in range(NBUF):
        e_start(jnp.int32(k))
    e_wait(jnp.int32(0))
    g_desc(jnp.int32(0)).start()

    @pl.loop(0, NCH)
    def _window(j):
        p = lax.rem(j, NBUF)
        q = lax.rem(j, NE)
        jn = j + 1

        @pl.when(jn < NCH)
        def _issue_next_gather():
            e_wait(jn)

            @pl.when(jn >= NBUF)
            def _():
                a_desc(jn - NBUF).wait()

            g_desc(jn).start()

        @pl.when(j + NBUF < NCH)
        def _prefetch_edges():
            e_start(j + NBUF)

        g_desc(j).wait()

        qsplat = jnp.full((16,), q, jnp.int32)

        @plsc.parallel_loop(0, CHUNK, 1, unroll=4)
        def _edge(r):
            wspl = plsc.load_gather(
                we, [qsplat, jnp.full((16,), r, jnp.int32)])
            for g in range(GROUPS):
                sl = pl.ds(g * 16, 16)
                rbuf[p, r, sl] = rbuf[p, r, sl] * wspl

        a_desc(j).start(add=True)

    for k in range(NBUF):
        a_desc(jnp.int32(NCH - NBUF + k)).wait()

    plsc.subcore_barrier()

    pltpu.sync_copy(acc.at[pl.ds(base, RPT)],
                    out_hbm.at[pl.ds(c * N + base, RPT)])

    @pl.when(s == NT - 1)
    def _write_tail():
        pltpu.sync_copy(acc.at[pl.ds(NT * RPT, RTAIL)],
                        out_hbm.at[pl.ds(c * N + NT * RPT, RTAIL)])


def _sc_compiler_params():
    cp = pltpu.CompilerParams()
    if "needs_layout_passes" in pltpu.CompilerParams.__dataclass_fields__:
        cp = dataclasses.replace(cp, needs_layout_passes=False)
    return cp


def _sc_agg(src_c, dst_t, w_t, h):
    mesh = plsc.VectorSubcoreMesh(core_axis_name="c", subcore_axis_name="s")
    kern = functools.partial(
        pl.kernel,
        mesh=mesh,
        compiler_params=_sc_compiler_params(),
        out_type=jax.ShapeDtypeStruct((2 * N, H), jnp.float32),
        scratch_types=[
            pltpu.VMEM((NE, CHUNK), jnp.int32),
            pltpu.VMEM((NE, CHUNK), jnp.int32),
            pltpu.VMEM((NE, CHUNK), jnp.float32),
            pltpu.VMEM((NBUF, CHUNK, H), jnp.float32),
            pltpu.VMEM_SHARED((N, H), jnp.float32),
            pltpu.SemaphoreType.DMA((NE,)),
            pltpu.SemaphoreType.DMA((NBUF,)),
            pltpu.SemaphoreType.DMA((NBUF,)),
        ],
    )(_sc_agg_kernel)
    return kern(src_c, dst_t, w_t, h)


@jax.jit
def _run(input, edge, symm_update, W1, b1, W2, b2):
    src = edge[0]
    dst = edge[1]
    src_c = jnp.stack([src, src + N]).reshape(2, NT, NCH, CHUNK)
    dst_t = dst.reshape(NT, NCH, CHUNK)
    w_t = symm_update.reshape(NT, NCH, CHUNK)

    h1 = _mm1(input, W1, b1)
    agg1 = _sc_agg(src_c, dst_t, w_t, h1)
    h2 = _mm2(agg1, W2, b2)
    agg2 = _sc_agg(src_c, dst_t, w_t, h2)
    return _epilogue(input, agg2)


def kernel(input, edge, symm_update, W1, b1, W2, b2):
    return _run(input, edge, symm_update, W1, b1, W2, b2)


# speedup vs baseline: 7.2070x; 1.0043x over previous
import dataclasses
import functools

import jax
import jax.numpy as jnp
from jax import lax
from jax.experimental import pallas as pl
from jax.experimental.pallas import tpu as pltpu
from jax.experimental.pallas import tpu_sc as plsc

N = 10000
E = 160000
C = 256
H = 128
NT = 16
EPT = E // NT
CHUNK = 80
NCH = EPT // CHUNK
RPT = 624
RTAIL = N - NT * RPT
GROUPS = H // 16

BN = 1000
NB = N // BN


def _mm1_body(x_ref, w_ref, b_ref, o_ref):
    o_ref[...] = (
        jnp.dot(x_ref[...], w_ref[...], preferred_element_type=jnp.float32)
        + b_ref[0]
    )


def _mm1(x, w, b):
    b2 = b.reshape(2, 1, H)
    return pl.pallas_call(
        _mm1_body,
        grid=(NB, 2),
        in_specs=[
            pl.BlockSpec((BN, C), lambda i, c: (i, 0)),
            pl.BlockSpec((C, H), lambda i, c: (0, c)),
            pl.BlockSpec((1, 1, H), lambda i, c: (c, 0, 0)),
        ],
        out_specs=pl.BlockSpec((BN, H), lambda i, c: (c * NB + i, 0)),
        out_shape=jax.ShapeDtypeStruct((2 * N, H), jnp.float32),
    )(x, w, b2)


def _mm2_body(a0_ref, a1_ref, w_ref, b_ref, o_ref):
    x0 = jnp.maximum(a0_ref[...], 0.0)
    x1 = jnp.maximum(a1_ref[...], 0.0)
    o_ref[...] = (
        jnp.dot(x0, w_ref[:H, :], preferred_element_type=jnp.float32)
        + jnp.dot(x1, w_ref[H:, :], preferred_element_type=jnp.float32)
        + b_ref[0]
    )


def _mm2(agg, w, b):
    b2 = b.reshape(2, 1, H)
    return pl.pallas_call(
        _mm2_body,
        grid=(NB, 2),
        in_specs=[
            pl.BlockSpec((BN, H), lambda i, c: (i, 0)),
            pl.BlockSpec((BN, H), lambda i, c: (NB + i, 0)),
            pl.BlockSpec((C, H), lambda i, c: (0, c)),
            pl.BlockSpec((1, 1, H), lambda i, c: (c, 0, 0)),
        ],
        out_specs=pl.BlockSpec((BN, H), lambda i, c: (c * NB + i, 0)),
        out_shape=jax.ShapeDtypeStruct((2 * N, H), jnp.float32),
    )(agg, agg, w, b2)


def _epilogue_body(in_ref, a0_ref, a1_ref, o_ref):
    o_ref[:, :H] = (in_ref[:, :H] + jnp.maximum(a0_ref[...], 0.0)) * 0.5
    o_ref[:, H:] = (in_ref[:, H:] + jnp.maximum(a1_ref[...], 0.0)) * 0.5


def _epilogue(inp, agg):
    return pl.pallas_call(
        _epilogue_body,
        grid=(NB,),
        in_specs=[
            pl.BlockSpec((BN, C), lambda i: (i, 0)),
            pl.BlockSpec((BN, H), lambda i: (i, 0)),
            pl.BlockSpec((BN, H), lambda i: (NB + i, 0)),
        ],
        out_specs=pl.BlockSpec((BN, C), lambda i: (i, 0)),
        out_shape=jax.ShapeDtypeStruct((N, C), jnp.float32),
    )(inp, agg, agg)


NBUF = 3
NE = 2 * NBUF


def _sc_agg_kernel(src_hbm, dst_hbm, w_hbm, h_hbm, out_hbm,
                   se, de, we, rbuf, acc, esem, gsem, asem):
    c = lax.axis_index("c")
    s = lax.axis_index("s")

    def e_copies(j):
        q = lax.rem(j, NE)
        return (
            pltpu.make_async_copy(src_hbm.at[c, s, j], se.at[q], esem.at[q]),
            pltpu.make_async_copy(dst_hbm.at[s, j], de.at[q], esem.at[q]),
            pltpu.make_async_copy(w_hbm.at[s, j], we.at[q], esem.at[q]),
        )

    def e_start(j):
        for cp in e_copies(j):
            cp.start()

    def e_wait(j):
        for cp in e_copies(j):
            cp.wait()

    def g_desc(j):
        p = lax.rem(j, NBUF)
        q = lax.rem(j, NE)
        return pltpu.make_async_copy(h_hbm.at[se.at[q]], rbuf.at[p],
                                     gsem.at[p])

    def a_desc(j):
        p = lax.rem(j, NBUF)
        q = lax.rem(j, NE)
        return pltpu.make_async_copy(rbuf.at[p], acc.at[de.at[q]],
                                     asem.at[p])

    zero = jnp.zeros((16,), jnp.float32)

    @pl.loop(0, CHUNK)
    def _zero_buf(r):
        for g in range(GROUPS):
            rbuf[0, r, pl.ds(g * 16, 16)] = zero

    base = s * RPT
    for k in range(RPT // CHUNK):
        pltpu.sync_copy(rbuf.at[0], acc.at[pl.ds(base + k * CHUNK, CHUNK)])
    rem = RPT % CHUNK
    if rem:
        pltpu.sync_copy(rbuf.at[0].at[pl.ds(0, rem)],
                        acc.at[pl.ds(base + (RPT // CHUNK) * CHUNK, rem)])

    @pl.when(s == NT - 1)
    def _zero_tail():
        pltpu.sync_copy(rbuf.at[0].at[pl.ds(0, RTAIL)],
                        acc.at[pl.ds(NT * RPT, RTAIL)])

    plsc.subcore_barrier()

    for k in range(NBUF):
        e_start(jnp.int32(k))
    e_wait(jnp.int32(0))
    g_desc(jnp.int32(0)).start()

    @pl.loop(0, NCH)
    def _window(j):
        p = lax.rem(j, NBUF)
        q = lax.rem(j, NE)
        jn = j + 1

        @pl.when(jn < NCH)
        def _issue_next_gather():
            e_wait(jn)

            @pl.when(jn >= NBUF)
            def _():
                a_desc(jn - NBUF).wait()

            g_desc(jn).start()

        @pl.when(j + NBUF < NCH)
        def _prefetch_edges():
            e_start(j + NBUF)

        g_desc(j).wait()

        qsplat = jnp.full((16,), q, jnp.int32)

        @plsc.parallel_loop(0, CHUNK, 1, unroll=8)
        def _edge(r):
            wspl = plsc.load_gather(
                we, [qsplat, jnp.full((16,), r, jnp.int32)])
            for g in range(GROUPS):
                sl = pl.ds(g * 16, 16)
                rbuf[p, r, sl] = rbuf[p, r, sl] * wspl

        a_desc(j).start(add=True)

    for k in range(NBUF):
        a_desc(jnp.int32(NCH - NBUF + k)).wait()

    plsc.subcore_barrier()

    pltpu.sync_copy(acc.at[pl.ds(base, RPT)],
                    out_hbm.at[pl.ds(c * N + base, RPT)])

    @pl.when(s == NT - 1)
    def _write_tail():
        pltpu.sync_copy(acc.at[pl.ds(NT * RPT, RTAIL)],
                        out_hbm.at[pl.ds(c * N + NT * RPT, RTAIL)])


def _sc_compiler_params():
    cp = pltpu.CompilerParams()
    if "needs_layout_passes" in pltpu.CompilerParams.__dataclass_fields__:
        cp = dataclasses.replace(cp, needs_layout_passes=False)
    return cp


def _sc_agg(src_c, dst_t, w_t, h):
    mesh = plsc.VectorSubcoreMesh(core_axis_name="c", subcore_axis_name="s")
    kern = functools.partial(
        pl.kernel,
        mesh=mesh,
        compiler_params=_sc_compiler_params(),
        out_type=jax.ShapeDtypeStruct((2 * N, H), jnp.float32),
        scratch_types=[
            pltpu.VMEM((NE, CHUNK), jnp.int32),
            pltpu.VMEM((NE, CHUNK), jnp.int32),
            pltpu.VMEM((NE, CHUNK), jnp.float32),
            pltpu.VMEM((NBUF, CHUNK, H), jnp.float32),
            pltpu.VMEM_SHARED((N, H), jnp.float32),
            pltpu.SemaphoreType.DMA((NE,)),
            pltpu.SemaphoreType.DMA((NBUF,)),
            pltpu.SemaphoreType.DMA((NBUF,)),
        ],
    )(_sc_agg_kernel)
    return kern(src_c, dst_t, w_t, h)


@jax.jit
def _run(input, edge, symm_update, W1, b1, W2, b2):
    src = edge[0]
    dst = edge[1]
    src_c = jnp.stack([src, src + N]).reshape(2, NT, NCH, CHUNK)
    dst_t = dst.reshape(NT, NCH, CHUNK)
    w_t = symm_update.reshape(NT, NCH, CHUNK)

    h1 = _mm1(input, W1, b1)
    agg1 = _sc_agg(src_c, dst_t, w_t, h1)
    h2 = _mm2(agg1, W2, b2)
    agg2 = _sc_agg(src_c, dst_t, w_t, h2)
    return _epilogue(input, agg2)


def kernel(input, edge, symm_update, W1, b1, W2, b2):
    return _run(input, edge, symm_update, W1, b1, W2, b2)
